# baseline (speedup 1.0000x reference)
import functools

import jax
import jax.numpy as jnp
from jax import lax
from jax.experimental import pallas as pl
from jax.experimental.pallas import tpu as pltpu
from jax.experimental.pallas import tpu_sc as plsc

_M, _N = 16384, 2048
_STRIDE = 16
_LANES = 16
_NC, _NS = 2, 16
_NW = _NC * _NS
_ROWS_PER_W = _M // _NW
_BR = 8
_NBLK = _ROWS_PER_W // _BR
_GROUPS = _N // (_LANES * _STRIDE)

_mesh = plsc.VectorSubcoreMesh(core_axis_name="c", subcore_axis_name="s")


def _copy_kept(in_b, out_b):
    col_base = lax.iota(jnp.int32, _LANES) * _STRIDE
    for r in range(_BR):
        row_idx = jnp.full((_LANES,), r, jnp.int32)
        for g in range(_GROUPS):
            col_idx = col_base + (g * _LANES * _STRIDE)
            v = plsc.load_gather(in_b, [row_idx, col_idx])
            plsc.store_scatter(out_b, [row_idx, col_idx], v)


@functools.partial(
    pl.kernel,
    out_type=jax.ShapeDtypeStruct((_M, _N), jnp.float32),
    mesh=_mesh,
    compiler_params=pltpu.CompilerParams(
        use_tc_tiling_on_sc=False, needs_layout_passes=False
    ),
    scratch_types=[
        pltpu.VMEM((_BR, _N), jnp.float32),
        pltpu.VMEM((_BR, _N), jnp.float32),
        pltpu.VMEM((_BR, _N), jnp.float32),
        pltpu.VMEM((_BR, _N), jnp.float32),
        pltpu.SemaphoreType.DMA,
        pltpu.SemaphoreType.DMA,
        pltpu.SemaphoreType.DMA,
        pltpu.SemaphoreType.DMA,
    ],
)
def _sc_sampler(x_hbm, o_hbm, in0, in1, out0, out1, si0, si1, so0, so1):
    wid = lax.axis_index("s") * _NC + lax.axis_index("c")
    row_base = wid * _ROWS_PER_W

    ins = (in0, in1)
    outs = (out0, out1)
    sis = (si0, si1)
    sos = (so0, so1)

    def rows_at(blk):
        return pl.ds(pl.multiple_of(row_base + blk * _BR, _BR), _BR)

    z = jnp.zeros((_LANES,), jnp.float32)
    for r in range(_BR):
        @pl.loop(0, _N // _LANES)
        def _zero(j):
            off = pl.multiple_of(j * _LANES, _LANES)
            out0[r, pl.ds(off, _LANES)] = z
            out1[r, pl.ds(off, _LANES)] = z

    pltpu.async_copy(x_hbm.at[rows_at(0)], in0, si0)
    pltpu.async_copy(x_hbm.at[rows_at(1)], in1, si1)

    @pl.loop(0, _NBLK, step=2)
    def _block(blk):
        for p in range(2):
            cur = blk + p
            in_b, out_b, si, so = ins[p], outs[p], sis[p], sos[p]

            pltpu.make_async_copy(x_hbm.at[rows_at(cur)], in_b, si).wait()

            @pl.when(cur >= 2)
            def _():
                pltpu.make_async_copy(out_b, o_hbm.at[rows_at(cur - 2)],
                                      so).wait()

            _copy_kept(in_b, out_b)
            pltpu.async_copy(out_b, o_hbm.at[rows_at(cur)], so)

            @pl.when(cur + 2 < _NBLK)
            def _():
                pltpu.async_copy(x_hbm.at[rows_at(cur + 2)], in_b, si)

    pltpu.make_async_copy(out0, o_hbm.at[rows_at(_NBLK - 2)], so0).wait()
    pltpu.make_async_copy(out1, o_hbm.at[rows_at(_NBLK - 1)], so1).wait()


def kernel(x):
    return _sc_sampler(x)


# speedup vs baseline: 2.9368x; 2.9368x over previous
import functools

import jax
import jax.numpy as jnp
from jax import lax
from jax.experimental import pallas as pl
from jax.experimental.pallas import tpu as pltpu
from jax.experimental.pallas import tpu_sc as plsc

_M, _N = 16384, 2048
_STRIDE = 16
_LANES = 16
_NC, _NS = 2, 16
_NW = _NC * _NS
_ROWS_PER_W = _M // _NW
_BR = 8
_NBLK = _ROWS_PER_W // _BR
_GROUPS = _N // (_LANES * _STRIDE)

_mesh = plsc.VectorSubcoreMesh(core_axis_name="c", subcore_axis_name="s")


def _copy_kept(in_b, out_b):
    col_base = lax.iota(jnp.int32, _LANES) * _STRIDE
    for r in range(_BR):
        row_idx = jnp.full((_LANES,), r, jnp.int32)
        for g in range(_GROUPS):
            col_idx = col_base + (g * _LANES * _STRIDE)
            v = plsc.load_gather(in_b, [row_idx, col_idx])
            plsc.store_scatter(out_b, [row_idx, col_idx], v)


@functools.partial(
    pl.kernel,
    out_type=jax.ShapeDtypeStruct((_M, _N), jnp.float32),
    mesh=_mesh,
    compiler_params=pltpu.CompilerParams(needs_layout_passes=False),
    scratch_types=[
        pltpu.VMEM((_BR, _N), jnp.float32),
        pltpu.VMEM((_BR, _N), jnp.float32),
        pltpu.VMEM((_BR, _N), jnp.float32),
        pltpu.VMEM((_BR, _N), jnp.float32),
        pltpu.SemaphoreType.DMA,
        pltpu.SemaphoreType.DMA,
        pltpu.SemaphoreType.DMA,
        pltpu.SemaphoreType.DMA,
    ],
)
def _sc_sampler(x_hbm, o_hbm, in0, in1, out0, out1, si0, si1, so0, so1):
    wid = lax.axis_index("s") * _NC + lax.axis_index("c")
    row_base = wid * _ROWS_PER_W

    ins = (in0, in1)
    outs = (out0, out1)
    sis = (si0, si1)
    sos = (so0, so1)

    def rows_at(blk):
        return pl.ds(pl.multiple_of(row_base + blk * _BR, _BR), _BR)

    z = jnp.zeros((_LANES,), jnp.float32)
    for r in range(_BR):
        @pl.loop(0, _N // _LANES)
        def _zero(j):
            off = pl.multiple_of(j * _LANES, _LANES)
            out0[r, pl.ds(off, _LANES)] = z
            out1[r, pl.ds(off, _LANES)] = z

    pltpu.async_copy(x_hbm.at[rows_at(0)], in0, si0)
    pltpu.async_copy(x_hbm.at[rows_at(1)], in1, si1)

    @pl.loop(0, _NBLK, step=2)
    def _block(blk):
        for p in range(2):
            cur = blk + p
            in_b, out_b, si, so = ins[p], outs[p], sis[p], sos[p]

            pltpu.make_async_copy(x_hbm.at[rows_at(cur)], in_b, si).wait()

            @pl.when(cur >= 2)
            def _():
                pltpu.make_async_copy(out_b, o_hbm.at[rows_at(cur - 2)],
                                      so).wait()

            _copy_kept(in_b, out_b)
            pltpu.async_copy(out_b, o_hbm.at[rows_at(cur)], so)

            @pl.when(cur + 2 < _NBLK)
            def _():
                pltpu.async_copy(x_hbm.at[rows_at(cur + 2)], in_b, si)

    pltpu.make_async_copy(out0, o_hbm.at[rows_at(_NBLK - 2)], so0).wait()
    pltpu.make_async_copy(out1, o_hbm.at[rows_at(_NBLK - 1)], so1).wait()


def kernel(x):
    return _sc_sampler(x)


# speedup vs baseline: 3.0532x; 1.0396x over previous
import functools

import jax
import jax.numpy as jnp
from jax import lax
from jax.experimental import pallas as pl
from jax.experimental.pallas import tpu as pltpu
from jax.experimental.pallas import tpu_sc as plsc

_M, _N = 16384, 2048
_STRIDE = 16
_LANES = 16
_NC, _NS = 2, 16
_NW = _NC * _NS
_ROWS_PER_W = _M // _NW
_BR = 4
_NBLK = _ROWS_PER_W // _BR
_RING = 4
_GROUPS = _N // (_LANES * _STRIDE)

_mesh = plsc.VectorSubcoreMesh(core_axis_name="c", subcore_axis_name="s")


def _copy_kept(in_b, out_b):
    col_base = lax.iota(jnp.int32, _LANES) * _STRIDE
    for r in range(_BR):
        row_idx = jnp.full((_LANES,), r, jnp.int32)
        for g in range(_GROUPS):
            col_idx = col_base + (g * _LANES * _STRIDE)
            v = plsc.load_gather(in_b, [row_idx, col_idx])
            plsc.store_scatter(out_b, [row_idx, col_idx], v)


@functools.partial(
    pl.kernel,
    out_type=jax.ShapeDtypeStruct((_M, _N), jnp.float32),
    mesh=_mesh,
    compiler_params=pltpu.CompilerParams(needs_layout_passes=False),
    scratch_types=(
        [pltpu.VMEM((_BR, _N), jnp.float32)] * (2 * _RING)
        + [pltpu.SemaphoreType.DMA] * (2 * _RING)
    ),
)
def _sc_sampler(x_hbm, o_hbm, *bufs):
    ins = bufs[:_RING]
    outs = bufs[_RING:2 * _RING]
    sis = bufs[2 * _RING:3 * _RING]
    sos = bufs[3 * _RING:4 * _RING]

    wid = lax.axis_index("s") * _NC + lax.axis_index("c")
    row_base = wid * _ROWS_PER_W

    def rows_at(blk):
        return pl.ds(pl.multiple_of(row_base + blk * _BR, _BR), _BR)

    z = jnp.zeros((_LANES,), jnp.float32)
    for r in range(_BR):
        @pl.loop(0, _N // _LANES)
        def _zero(j):
            off = pl.multiple_of(j * _LANES, _LANES)
            for p in range(_RING):
                outs[p][r, pl.ds(off, _LANES)] = z

    for p in range(_RING):
        pltpu.async_copy(x_hbm.at[rows_at(p)], ins[p], sis[p])

    @pl.loop(0, _NBLK, step=_RING)
    def _block(blk):
        for p in range(_RING):
            cur = blk + p
            in_b, out_b, si, so = ins[p], outs[p], sis[p], sos[p]

            pltpu.make_async_copy(x_hbm.at[rows_at(cur)], in_b, si).wait()

            @pl.when(cur >= _RING)
            def _():
                pltpu.make_async_copy(out_b, o_hbm.at[rows_at(cur - _RING)],
                                      so).wait()

            _copy_kept(in_b, out_b)
            pltpu.async_copy(out_b, o_hbm.at[rows_at(cur)], so)

            @pl.when(cur + _RING < _NBLK)
            def _():
                pltpu.async_copy(x_hbm.at[rows_at(cur + _RING)], in_b, si)

    for p in range(_RING):
        pltpu.make_async_copy(outs[p], o_hbm.at[rows_at(_NBLK - _RING + p)],
                              sos[p]).wait()


def kernel(x):
    return _sc_sampler(x)


# speedup vs baseline: 3.1010x; 1.0157x over previous
import functools

import jax
import jax.numpy as jnp
from jax import lax
from jax.experimental import pallas as pl
from jax.experimental.pallas import tpu as pltpu
from jax.experimental.pallas import tpu_sc as plsc

_M, _N = 16384, 2048
_STRIDE = 16
_LANES = 16
_NC, _NS = 2, 16
_NW = _NC * _NS
_ROWS_PER_W = _M // _NW
_BR = 4
_NBLK = _ROWS_PER_W // _BR
_RING = 4
_GROUPS = _N // (_LANES * _STRIDE)

_mesh = plsc.VectorSubcoreMesh(core_axis_name="c", subcore_axis_name="s")


def _copy_kept(in_b, out_b):
    kmask = jnp.where(lax.iota(jnp.int32, _LANES) == 0,
                      jnp.float32(1), jnp.float32(0))
    for r in range(_BR):
        @plsc.parallel_loop(0, _N // _LANES, unroll=8)
        def _mm(j):
            off = pl.multiple_of(j * _LANES, _LANES)
            out_b[r, pl.ds(off, _LANES)] = in_b[r, pl.ds(off, _LANES)] * kmask


@functools.partial(
    pl.kernel,
    out_type=jax.ShapeDtypeStruct((_M, _N), jnp.float32),
    mesh=_mesh,
    compiler_params=pltpu.CompilerParams(needs_layout_passes=False),
    scratch_types=(
        [pltpu.VMEM((_BR, _N), jnp.float32)] * (2 * _RING)
        + [pltpu.SemaphoreType.DMA] * (2 * _RING)
    ),
)
def _sc_sampler(x_hbm, o_hbm, *bufs):
    ins = bufs[:_RING]
    outs = bufs[_RING:2 * _RING]
    sis = bufs[2 * _RING:3 * _RING]
    sos = bufs[3 * _RING:4 * _RING]

    wid = lax.axis_index("s") * _NC + lax.axis_index("c")
    row_base = wid * _ROWS_PER_W

    def rows_at(blk):
        return pl.ds(pl.multiple_of(row_base + blk * _BR, _BR), _BR)

    for p in range(_RING):
        pltpu.async_copy(x_hbm.at[rows_at(p)], ins[p], sis[p])

    @pl.loop(0, _NBLK, step=_RING)
    def _block(blk):
        for p in range(_RING):
            cur = blk + p
            in_b, out_b, si, so = ins[p], outs[p], sis[p], sos[p]

            pltpu.make_async_copy(x_hbm.at[rows_at(cur)], in_b, si).wait()

            @pl.when(cur >= _RING)
            def _():
                pltpu.make_async_copy(out_b, o_hbm.at[rows_at(cur - _RING)],
                                      so).wait()

            _copy_kept(in_b, out_b)
            pltpu.async_copy(out_b, o_hbm.at[rows_at(cur)], so)

            @pl.when(cur + _RING < _NBLK)
            def _():
                pltpu.async_copy(x_hbm.at[rows_at(cur + _RING)], in_b, si)

    for p in range(_RING):
        pltpu.make_async_copy(outs[p], o_hbm.at[rows_at(_NBLK - _RING + p)],
                              sos[p]).wait()


def kernel(x):
    return _sc_sampler(x)


# speedup vs baseline: 3.1069x; 1.0019x over previous
import functools

import jax
import jax.numpy as jnp
from jax import lax
from jax.experimental import pallas as pl
from jax.experimental.pallas import tpu as pltpu
from jax.experimental.pallas import tpu_sc as plsc

_M, _N = 16384, 2048
_STRIDE = 16
_LANES = 16
_NC, _NS = 2, 16
_NW = _NC * _NS
_ROWS_PER_W = _M // _NW
_BR = 2
_NBLK = _ROWS_PER_W // _BR
_RING = 8
_GROUPS = _N // (_LANES * _STRIDE)

_mesh = plsc.VectorSubcoreMesh(core_axis_name="c", subcore_axis_name="s")


def _copy_kept(in_b, out_b):
    kmask = jnp.where(lax.iota(jnp.int32, _LANES) == 0,
                      jnp.float32(1), jnp.float32(0))
    for r in range(_BR):
        @plsc.parallel_loop(0, _N // _LANES, unroll=8)
        def _mm(j):
            off = pl.multiple_of(j * _LANES, _LANES)
            out_b[r, pl.ds(off, _LANES)] = in_b[r, pl.ds(off, _LANES)] * kmask


@functools.partial(
    pl.kernel,
    out_type=jax.ShapeDtypeStruct((_M, _N), jnp.float32),
    mesh=_mesh,
    compiler_params=pltpu.CompilerParams(needs_layout_passes=False),
    scratch_types=(
        [pltpu.VMEM((_BR, _N), jnp.float32)] * (2 * _RING)
        + [pltpu.SemaphoreType.DMA] * (2 * _RING)
    ),
)
def _sc_sampler(x_hbm, o_hbm, *bufs):
    ins = bufs[:_RING]
    outs = bufs[_RING:2 * _RING]
    sis = bufs[2 * _RING:3 * _RING]
    sos = bufs[3 * _RING:4 * _RING]

    wid = lax.axis_index("s") * _NC + lax.axis_index("c")
    row_base = wid * _ROWS_PER_W

    def rows_at(blk):
        return pl.ds(pl.multiple_of(row_base + blk * _BR, _BR), _BR)

    for p in range(_RING):
        pltpu.async_copy(x_hbm.at[rows_at(p)], ins[p], sis[p])

    @pl.loop(0, _NBLK, step=_RING)
    def _block(blk):
        for p in range(_RING):
            cur = blk + p
            in_b, out_b, si, so = ins[p], outs[p], sis[p], sos[p]

            pltpu.make_async_copy(x_hbm.at[rows_at(cur)], in_b, si).wait()

            @pl.when(cur >= _RING)
            def _():
                pltpu.make_async_copy(out_b, o_hbm.at[rows_at(cur - _RING)],
                                      so).wait()

            _copy_kept(in_b, out_b)
            pltpu.async_copy(out_b, o_hbm.at[rows_at(cur)], so)

            @pl.when(cur + _RING < _NBLK)
            def _():
                pltpu.async_copy(x_hbm.at[rows_at(cur + _RING)], in_b, si)

    for p in range(_RING):
        pltpu.make_async_copy(outs[p], o_hbm.at[rows_at(_NBLK - _RING + p)],
                              sos[p]).wait()


def kernel(x):
    return _sc_sampler(x)


# speedup vs baseline: 3.1141x; 1.0023x over previous
import functools

import jax
import jax.numpy as jnp
from jax import lax
from jax.experimental import pallas as pl
from jax.experimental.pallas import tpu as pltpu
from jax.experimental.pallas import tpu_sc as plsc

_M, _N = 16384, 2048
_STRIDE = 16
_LANES = 16
_NC, _NS = 2, 16
_NW = _NC * _NS
_ROWS_PER_W = _M // _NW
_BR = 2
_NBLK = _ROWS_PER_W // _BR
_RING = 8
_GROUPS = _N // (_LANES * _STRIDE)

_mesh = plsc.VectorSubcoreMesh(core_axis_name="c", subcore_axis_name="s")


def _copy_kept(in_b, out_b):
    kmask = jnp.where(lax.iota(jnp.int32, _LANES) == 0,
                      jnp.float32(1), jnp.float32(0))
    for r in range(_BR):
        @plsc.parallel_loop(0, _N // _LANES, unroll=8)
        def _mm(j):
            off = pl.multiple_of(j * _LANES, _LANES)
            out_b[r, pl.ds(off, _LANES)] = in_b[r, pl.ds(off, _LANES)] * kmask


@functools.partial(
    pl.kernel,
    out_type=jax.ShapeDtypeStruct((_M, _N), jnp.float32),
    mesh=_mesh,
    compiler_params=pltpu.CompilerParams(
        needs_layout_passes=False,
        disable_bounds_checks=True,
        disable_semaphore_checks=True,
    ),
    scratch_types=(
        [pltpu.VMEM((_BR, _N), jnp.float32)] * (2 * _RING)
        + [pltpu.SemaphoreType.DMA] * (2 * _RING)
    ),
)
def _sc_sampler(x_hbm, o_hbm, *bufs):
    ins = bufs[:_RING]
    outs = bufs[_RING:2 * _RING]
    sis = bufs[2 * _RING:3 * _RING]
    sos = bufs[3 * _RING:4 * _RING]

    wid = lax.axis_index("s") * _NC + lax.axis_index("c")
    row_base = wid * _ROWS_PER_W

    def rows_at(blk):
        return pl.ds(pl.multiple_of(row_base + blk * _BR, _BR), _BR)

    for p in range(_RING):
        pltpu.async_copy(x_hbm.at[rows_at(p)], ins[p], sis[p])

    @pl.loop(0, _NBLK, step=_RING)
    def _block(blk):
        for p in range(_RING):
            cur = blk + p
            in_b, out_b, si, so = ins[p], outs[p], sis[p], sos[p]

            pltpu.make_async_copy(x_hbm.at[rows_at(cur)], in_b, si).wait()

            @pl.when(cur >= _RING)
            def _():
                pltpu.make_async_copy(out_b, o_hbm.at[rows_at(cur - _RING)],
                                      so).wait()

            _copy_kept(in_b, out_b)
            pltpu.async_copy(out_b, o_hbm.at[rows_at(cur)], so)

            @pl.when(cur + _RING < _NBLK)
            def _():
                pltpu.async_copy(x_hbm.at[rows_at(cur + _RING)], in_b, si)

    for p in range(_RING):
        pltpu.make_async_copy(outs[p], o_hbm.at[rows_at(_NBLK - _RING + p)],
                              sos[p]).wait()


def kernel(x):
    return _sc_sampler(x)


# speedup vs baseline: 3.1169x; 1.0009x over previous
import functools

import jax
import jax.numpy as jnp
from jax import lax
from jax.experimental import pallas as pl
from jax.experimental.pallas import tpu as pltpu
from jax.experimental.pallas import tpu_sc as plsc

_M, _N = 16384, 2048
_STRIDE = 16
_LANES = 16
_NC, _NS = 2, 16
_NW = _NC * _NS
_ROWS_PER_W = _M // _NW
_BR = 2
_NBLK = _ROWS_PER_W // _BR
_RING = 8
_GROUPS = _N // (_LANES * _STRIDE)

_mesh = plsc.VectorSubcoreMesh(core_axis_name="c", subcore_axis_name="s")


def _copy_kept(in_b, out_b):
    kmask = jnp.where(lax.iota(jnp.int32, _LANES) == 0,
                      jnp.float32(1), jnp.float32(0))
    for r in range(_BR):
        @plsc.parallel_loop(0, _N // _LANES, unroll=8)
        def _mm(j):
            off = pl.multiple_of(j * _LANES, _LANES)
            out_b[r, pl.ds(off, _LANES)] = in_b[r, pl.ds(off, _LANES)] * kmask


@functools.partial(
    pl.kernel,
    out_type=jax.ShapeDtypeStruct((_M, _N), jnp.float32),
    mesh=_mesh,
    compiler_params=pltpu.CompilerParams(
        needs_layout_passes=False,
        disable_bounds_checks=True,
        disable_semaphore_checks=True,
        skip_device_barrier=True,
    ),
    scratch_types=(
        [pltpu.VMEM((_BR, _N), jnp.float32)] * (2 * _RING)
        + [pltpu.SemaphoreType.DMA] * (2 * _RING)
    ),
)
def _sc_sampler(x_hbm, o_hbm, *bufs):
    ins = bufs[:_RING]
    outs = bufs[_RING:2 * _RING]
    sis = bufs[2 * _RING:3 * _RING]
    sos = bufs[3 * _RING:4 * _RING]

    wid = lax.axis_index("s") * _NC + lax.axis_index("c")
    row_base = wid * _ROWS_PER_W

    def rows_at(blk):
        return pl.ds(pl.multiple_of(row_base + blk * _BR, _BR), _BR)

    for p in range(_RING):
        pltpu.async_copy(x_hbm.at[rows_at(p)], ins[p], sis[p])

    @pl.loop(0, _NBLK, step=_RING)
    def _block(blk):
        for p in range(_RING):
            cur = blk + p
            in_b, out_b, si, so = ins[p], outs[p], sis[p], sos[p]

            pltpu.make_async_copy(x_hbm.at[rows_at(cur)], in_b, si).wait()

            @pl.when(cur >= _RING)
            def _():
                pltpu.make_async_copy(out_b, o_hbm.at[rows_at(cur - _RING)],
                                      so).wait()

            _copy_kept(in_b, out_b)
            pltpu.async_copy(out_b, o_hbm.at[rows_at(cur)], so)

            @pl.when(cur + _RING < _NBLK)
            def _():
                pltpu.async_copy(x_hbm.at[rows_at(cur + _RING)], in_b, si)

    for p in range(_RING):
        pltpu.make_async_copy(outs[p], o_hbm.at[rows_at(_NBLK - _RING + p)],
                              sos[p]).wait()


def kernel(x):
    return _sc_sampler(x)
